# baseline (speedup 1.0000x reference)
import functools

import jax
import jax.numpy as jnp
from jax import lax
from jax.experimental import pallas as pl
from jax.experimental.pallas import tpu as pltpu
from jax.experimental.pallas import tpu_sc as plsc

B = 16384
EMB = 32
NW = 32
ROWS_PER_W = B // NW
CHUNK = 128
NCHUNK = ROWS_PER_W // CHUNK


@functools.cache
def _build_sc_gather():
    mesh = plsc.VectorSubcoreMesh(core_axis_name="c", subcore_axis_name="s")

    @functools.partial(
        pl.kernel,
        mesh=mesh,
        compiler_params=pltpu.CompilerParams(use_tc_tiling_on_sc=False),
        out_type=(
            jax.ShapeDtypeStruct((B, EMB), jnp.float32),
            jax.ShapeDtypeStruct((B, EMB), jnp.float32),
        ),
        scratch_types=[
            pltpu.VMEM((NCHUNK, CHUNK), jnp.int32),
            pltpu.VMEM((NCHUNK, CHUNK), jnp.int32),
            pltpu.VMEM((ROWS_PER_W, EMB), jnp.float32),
            pltpu.VMEM((ROWS_PER_W, EMB), jnp.float32),
            pltpu.SemaphoreType.DMA,
        ],
    )
    def sc_gather(uid_hbm, iid_hbm, utab_hbm, itab_hbm, uout_hbm, iout_hbm,
                  uidx_v, iidx_v, urows_v, irows_v, sem):
        wid = lax.axis_index("s") * 2 + lax.axis_index("c")
        pltpu.sync_copy(uid_hbm.at[pl.ds(wid * NCHUNK, NCHUNK)], uidx_v)
        pltpu.sync_copy(iid_hbm.at[pl.ds(wid * NCHUNK, NCHUNK)], iidx_v)
        copies = []
        for j in range(NCHUNK):
            copies.append(pltpu.async_copy(
                utab_hbm.at[uidx_v.at[j]],
                urows_v.at[pl.ds(j * CHUNK, CHUNK)], sem))
        for j in range(NCHUNK):
            copies.append(pltpu.async_copy(
                itab_hbm.at[iidx_v.at[j]],
                irows_v.at[pl.ds(j * CHUNK, CHUNK)], sem))
        for c in copies:
            c.wait()
        base = wid * ROWS_PER_W
        pltpu.sync_copy(urows_v, uout_hbm.at[pl.ds(base, ROWS_PER_W)])
        pltpu.sync_copy(irows_v, iout_hbm.at[pl.ds(base, ROWS_PER_W)])

    return sc_gather


def _mlp_body(u_ref, v_ref, w0a, w0b, b0, w1, b1, w2, b2, w3, b3, wout, bout,
              o_ref):
    dot = functools.partial(jnp.dot, preferred_element_type=jnp.float32)
    x = jnp.maximum(dot(u_ref[...], w0a[...]) + dot(v_ref[...], w0b[...])
                    + b0[...], 0.0)
    x = jnp.maximum(dot(x, w1[...]) + b1[...], 0.0)
    x = jnp.maximum(dot(x, w2[...]) + b2[...], 0.0)
    x = jnp.maximum(dot(x, w3[...]) + b3[...], 0.0)
    o_ref[...] = jax.nn.sigmoid(dot(x, wout[...]) + bout[...])


def _mlp(u, v, w0a, w0b, b0, w1, b1, w2, b2, w3, b3, wout, bout):
    blk = 2048
    grid = (B // blk,)

    def full(shape):
        return pl.BlockSpec(shape, lambda i: (0, 0))

    return pl.pallas_call(
        _mlp_body,
        grid=grid,
        in_specs=[
            pl.BlockSpec((blk, EMB), lambda i: (i, 0)),
            pl.BlockSpec((blk, EMB), lambda i: (i, 0)),
            full((EMB, 64)), full((EMB, 64)), full((1, 64)),
            full((64, 32)), full((1, 32)),
            full((32, 16)), full((1, 16)),
            full((16, 8)), full((1, 8)),
            full((8, 1)), full((1, 1)),
        ],
        out_specs=pl.BlockSpec((blk, 1), lambda i: (i, 0)),
        out_shape=jax.ShapeDtypeStruct((B, 1), jnp.float32),
    )(u, v, w0a, w0b, b0, w1, b1, w2, b2, w3, b3, wout, bout)


def kernel(user_id, item_id, user_table, item_table, W0, b0, W1, b1, W2, b2,
           W3, b3, Wout, bout):
    uid2 = user_id.astype(jnp.int32).reshape(B // CHUNK, CHUNK)
    iid2 = item_id.astype(jnp.int32).reshape(B // CHUNK, CHUNK)
    u_emb, i_emb = _build_sc_gather()(uid2, iid2, user_table, item_table)
    return _mlp(u_emb, i_emb, W0[:EMB], W0[EMB:], b0.reshape(1, -1),
                W1, b1.reshape(1, -1), W2, b2.reshape(1, -1),
                W3, b3.reshape(1, -1), Wout, bout.reshape(1, -1))
